# Initial kernel scaffold; baseline (speedup 1.0000x reference)
import jax
import jax.numpy as jnp
from jax.experimental import pallas as pl


def kernel(pos, x, idx_neighboors, K_points, W, bn_gamma, bn_beta):
    raise NotImplementedError("write your pallas kernel here")


# speedup vs baseline: 2.2270x; 2.2270x over previous
import functools

import jax
import jax.numpy as jnp
from jax import lax
from jax.experimental import pallas as pl
from jax.experimental.pallas import tpu as pltpu
from jax.experimental.pallas import tpu_sc as plsc

N = 10000
H = 32
DIN = 128
DOUT = 128
KP = 15
KPP = 16
NH = N * H
PW = 128
BN_EPS = 1e-5
NEG_SLOPE = 0.2

B = 400
WIN = 128


def _sc_gather(feats_t, ptab, idx_flat):
    mesh = plsc.VectorSubcoreMesh(core_axis_name="c", subcore_axis_name="s")

    @functools.partial(
        pl.kernel,
        out_type=(
            jax.ShapeDtypeStruct((NH, DIN), jnp.float32),
            jax.ShapeDtypeStruct((NH, PW), jnp.float32),
        ),
        mesh=mesh,
    )
    def k(feat_hbm, ptab_hbm, idx_hbm, g_hbm, p_hbm):
        def body(i_vmem, g_vmem, p_vmem):
            pltpu.sync_copy(feat_hbm.at[i_vmem.at[0]], g_vmem)
            pltpu.sync_copy(ptab_hbm.at[i_vmem.at[0]], p_vmem)

        pltpu.emit_pipeline(
            body,
            grid=(NH // WIN,),
            in_specs=[pl.BlockSpec((1, WIN), lambda i: (0, i))],
            out_specs=[
                pl.BlockSpec((WIN, DIN), lambda i: (i, 0)),
                pl.BlockSpec((WIN, PW), lambda i: (i, 0)),
            ],
            core_axis_name=("c", "s"),
            dimension_semantics=(pltpu.PARALLEL,),
        )(idx_hbm, g_hbm, p_hbm)

    return k(feats_t, ptab, idx_flat)


def _main_body(g_ref, p_ref, q_ref, k_ref, w_ref, out_ref, stats_ref, acc_ref):
    i = pl.program_id(0)
    nb = pl.num_programs(0)
    g = g_ref[...].reshape(B, H, DIN)
    s = p_ref[...].reshape(B, H, PW)
    snorm = s[:, :, 3:4]
    q3 = q_ref[...][:, 0:3]
    k3 = k_ref[...]
    c = q3[:, None, :] + k3[None, :, :]
    cnorm = jnp.sum(c * c, axis=-1)
    dots = (
        s[:, :, 0:1] * c[:, None, :, 0]
        + s[:, :, 1:2] * c[:, None, :, 1]
        + s[:, :, 2:3] * c[:, None, :, 2]
    )
    sq = snorm - 2.0 * dots + cnorm[:, None, :]
    w = jnp.maximum(1.0 - jnp.sqrt(jnp.maximum(sq, 0.0)), 0.0)
    wf = jax.lax.dot_general(
        w, g, (((1,), (1,)), ((0,), (0,))),
        preferred_element_type=jnp.float32,
    )
    acc = jnp.zeros((B, DOUT), jnp.float32)
    for kk in range(KP):
        acc = acc + jnp.dot(wf[:, kk, :], w_ref[kk],
                            preferred_element_type=jnp.float32)
    out_ref[...] = acc

    @pl.when(i == 0)
    def _():
        acc_ref[...] = jnp.zeros_like(acc_ref)

    acc_ref[0:1, :] += jnp.sum(acc, axis=0, keepdims=True)
    acc_ref[1:2, :] += jnp.sum(acc * acc, axis=0, keepdims=True)

    @pl.when(i == nb - 1)
    def _():
        stats_ref[...] = acc_ref[...]


def _tc_main(gath, ptab_g, posq, k3p, W):
    nb = N // B
    return pl.pallas_call(
        _main_body,
        grid=(nb,),
        in_specs=[
            pl.BlockSpec((B * H, DIN), lambda i: (i, 0)),
            pl.BlockSpec((B * H, PW), lambda i: (i, 0)),
            pl.BlockSpec((B, PW), lambda i: (i, 0)),
            pl.BlockSpec((KPP, 3), lambda i: (0, 0)),
            pl.BlockSpec((KP, DIN, DOUT), lambda i: (0, 0, 0)),
        ],
        out_specs=[
            pl.BlockSpec((B, DOUT), lambda i: (i, 0)),
            pl.BlockSpec((8, DOUT), lambda i: (0, 0)),
        ],
        out_shape=[
            jax.ShapeDtypeStruct((N, DOUT), jnp.float32),
            jax.ShapeDtypeStruct((8, DOUT), jnp.float32),
        ],
        scratch_shapes=[pltpu.VMEM((8, DOUT), jnp.float32)],
    )(gath, ptab_g, posq, k3p, W)


def _bn_body(o_ref, st_ref, ga_ref, be_ref, out_ref):
    inv_n = 1.0 / N
    mean = st_ref[0:1, :] * inv_n
    var = st_ref[1:2, :] * inv_n - mean * mean
    inv = jax.lax.rsqrt(var + BN_EPS)
    y = (o_ref[...] - mean) * (inv * ga_ref[...]) + be_ref[...]
    out_ref[...] = jnp.where(y >= 0.0, y, NEG_SLOPE * y)


def _tc_bn(out_raw, stats, gamma, beta):
    b2 = 2000
    return pl.pallas_call(
        _bn_body,
        grid=(N // b2,),
        in_specs=[
            pl.BlockSpec((b2, DOUT), lambda i: (i, 0)),
            pl.BlockSpec((8, DOUT), lambda i: (0, 0)),
            pl.BlockSpec((1, DOUT), lambda i: (0, 0)),
            pl.BlockSpec((1, DOUT), lambda i: (0, 0)),
        ],
        out_specs=pl.BlockSpec((b2, DOUT), lambda i: (i, 0)),
        out_shape=jax.ShapeDtypeStruct((N, DOUT), jnp.float32),
    )(out_raw, stats, gamma, beta)


def kernel(pos, x, idx_neighboors, K_points, W, bn_gamma, bn_beta):
    idx_i32 = idx_neighboors.astype(jnp.int32)
    idx_flat = idx_i32.reshape(1, NH)
    feats_t = jnp.concatenate(
        [x, jnp.zeros((1, DIN), x.dtype)], axis=0)
    pos_t = jnp.concatenate(
        [pos, jnp.full((1, 3), 1e6, pos.dtype)], axis=0)
    pnorm = jnp.sum(pos_t * pos_t, axis=1, keepdims=True)
    ptab = jnp.concatenate(
        [pos_t, pnorm, jnp.zeros((N + 1, PW - 4), jnp.float32)], axis=1)
    posq = jnp.pad(pos, ((0, 0), (0, PW - 3)))
    k3p = jnp.concatenate(
        [K_points, jnp.full((KPP - KP, 3), 1e6, jnp.float32)], axis=0)

    gath, ptab_g = _sc_gather(feats_t, ptab, idx_flat)
    out_raw, stats = _tc_main(gath, ptab_g, posq, k3p, W)
    return _tc_bn(out_raw, stats, bn_gamma.reshape(1, DOUT),
                  bn_beta.reshape(1, DOUT))


# speedup vs baseline: 2.5325x; 1.1372x over previous
import functools

import jax
import jax.numpy as jnp
from jax import lax
from jax.experimental import pallas as pl
from jax.experimental.pallas import tpu as pltpu
from jax.experimental.pallas import tpu_sc as plsc

N = 10000
H = 32
DIN = 128
DOUT = 128
KP = 15
KPP = 16
NH = N * H
TW = 128
FH = 120
BN_EPS = 1e-5
NEG_SLOPE = 0.2

B = 400
WIN = 128


def _sc_gather(tab, idx_flat):
    mesh = plsc.VectorSubcoreMesh(core_axis_name="c", subcore_axis_name="s")

    @functools.partial(
        pl.kernel,
        out_type=jax.ShapeDtypeStruct((NH, TW), jnp.float32),
        mesh=mesh,
    )
    def k(tab_hbm, idx_hbm, g_hbm):
        def body(i_vmem, g_vmem):
            pltpu.sync_copy(tab_hbm.at[i_vmem.at[0]], g_vmem)

        pltpu.emit_pipeline(
            body,
            grid=(NH // WIN,),
            in_specs=[pl.BlockSpec((1, WIN), lambda i: (0, i))],
            out_specs=[pl.BlockSpec((WIN, TW), lambda i: (i, 0))],
            core_axis_name=("c", "s"),
            dimension_semantics=(pltpu.PARALLEL,),
        )(idx_hbm, g_hbm)

    return k(tab, idx_flat)


def _main_body(g_ref, q_ref, k_ref, w_ref, out_ref, stats_ref, acc_ref):
    i = pl.program_id(0)
    nb = pl.num_programs(0)
    gfull = g_ref[...]
    packed = lax.bitcast_convert_type(gfull[:, FH:FH + 4], jnp.uint32)
    f_hi = lax.bitcast_convert_type(
        packed & jnp.uint32(0xFFFF0000), jnp.float32)
    f_lo = lax.bitcast_convert_type(
        packed << 16, jnp.float32)
    g = jnp.concatenate([gfull[:, 0:FH], f_hi, f_lo], axis=1)
    gf = g.reshape(B, H, DIN)
    s = gfull[:, FH + 4:TW].reshape(B, H, 4)
    snorm = s[:, :, 3:4]
    q3 = q_ref[...][:, 0:3]
    k3 = k_ref[...]
    c = q3[:, None, :] + k3[None, :, :]
    cnorm = jnp.sum(c * c, axis=-1)
    dots = (
        s[:, :, 0:1] * c[:, None, :, 0]
        + s[:, :, 1:2] * c[:, None, :, 1]
        + s[:, :, 2:3] * c[:, None, :, 2]
    )
    sq = snorm - 2.0 * dots + cnorm[:, None, :]
    w = jnp.maximum(1.0 - jnp.sqrt(jnp.maximum(sq, 0.0)), 0.0)
    wf = lax.dot_general(
        w, gf, (((1,), (1,)), ((0,), (0,))),
        preferred_element_type=jnp.float32,
    )
    acc = jnp.zeros((B, DOUT), jnp.float32)
    for kk in range(KP):
        acc = acc + jnp.dot(wf[:, kk, :], w_ref[kk],
                            preferred_element_type=jnp.float32)
    out_ref[...] = acc

    @pl.when(i == 0)
    def _():
        acc_ref[...] = jnp.zeros_like(acc_ref)

    acc_ref[0:1, :] += jnp.sum(acc, axis=0, keepdims=True)
    acc_ref[1:2, :] += jnp.sum(acc * acc, axis=0, keepdims=True)

    @pl.when(i == nb - 1)
    def _():
        stats_ref[...] = acc_ref[...]


def _tc_main(gath, posq, k3p, W):
    nb = N // B
    return pl.pallas_call(
        _main_body,
        grid=(nb,),
        in_specs=[
            pl.BlockSpec((B * H, TW), lambda i: (i, 0)),
            pl.BlockSpec((B, 8), lambda i: (i, 0)),
            pl.BlockSpec((KPP, 3), lambda i: (0, 0)),
            pl.BlockSpec((KP, DIN, DOUT), lambda i: (0, 0, 0)),
        ],
        out_specs=[
            pl.BlockSpec((B, DOUT), lambda i: (i, 0)),
            pl.BlockSpec((8, DOUT), lambda i: (0, 0)),
        ],
        out_shape=[
            jax.ShapeDtypeStruct((N, DOUT), jnp.float32),
            jax.ShapeDtypeStruct((8, DOUT), jnp.float32),
        ],
        scratch_shapes=[pltpu.VMEM((8, DOUT), jnp.float32)],
    )(gath, posq, k3p, W)


def _bn_body(o_ref, st_ref, ga_ref, be_ref, out_ref):
    inv_n = 1.0 / N
    mean = st_ref[0:1, :] * inv_n
    var = st_ref[1:2, :] * inv_n - mean * mean
    inv = lax.rsqrt(var + BN_EPS)
    y = (o_ref[...] - mean) * (inv * ga_ref[...]) + be_ref[...]
    out_ref[...] = jnp.where(y >= 0.0, y, NEG_SLOPE * y)


def _tc_bn(out_raw, stats, gamma, beta):
    b2 = 2000
    return pl.pallas_call(
        _bn_body,
        grid=(N // b2,),
        in_specs=[
            pl.BlockSpec((b2, DOUT), lambda i: (i, 0)),
            pl.BlockSpec((8, DOUT), lambda i: (0, 0)),
            pl.BlockSpec((1, DOUT), lambda i: (0, 0)),
            pl.BlockSpec((1, DOUT), lambda i: (0, 0)),
        ],
        out_specs=pl.BlockSpec((b2, DOUT), lambda i: (i, 0)),
        out_shape=jax.ShapeDtypeStruct((N, DOUT), jnp.float32),
    )(out_raw, stats, gamma, beta)


def kernel(pos, x, idx_neighboors, K_points, W, bn_gamma, bn_beta):
    idx_flat = idx_neighboors.astype(jnp.int32).reshape(1, NH)
    pos_t = jnp.concatenate(
        [pos, jnp.full((1, 3), 1e6, pos.dtype)], axis=0)
    pnorm = jnp.sum(pos_t * pos_t, axis=1, keepdims=True)
    p4 = jnp.concatenate([pos_t, pnorm], axis=1)
    feats_t = jnp.concatenate(
        [x, jnp.zeros((1, DIN), x.dtype)], axis=0)
    t1 = lax.bitcast_convert_type(
        feats_t[:, FH:FH + 4].astype(jnp.bfloat16), jnp.uint16)
    t2 = lax.bitcast_convert_type(
        feats_t[:, FH + 4:DIN].astype(jnp.bfloat16), jnp.uint16)
    packed = (t1.astype(jnp.uint32) << 16) | t2.astype(jnp.uint32)
    tab = jnp.concatenate(
        [feats_t[:, 0:FH], lax.bitcast_convert_type(packed, jnp.float32), p4],
        axis=1)
    posq = jnp.pad(pos, ((0, 0), (0, 5)))
    k3p = jnp.concatenate(
        [K_points, jnp.full((KPP - KP, 3), 1e6, jnp.float32)], axis=0)

    gath = _sc_gather(tab, idx_flat)
    out_raw, stats = _tc_main(gath, posq, k3p, W)
    return _tc_bn(out_raw, stats, bn_gamma.reshape(1, DOUT),
                  bn_beta.reshape(1, DOUT))


# speedup vs baseline: 2.9038x; 1.1466x over previous
import functools

import jax
import jax.numpy as jnp
from jax import lax
from jax.experimental import pallas as pl
from jax.experimental.pallas import tpu as pltpu
from jax.experimental.pallas import tpu_sc as plsc

N = 10000
H = 32
DIN = 128
DOUT = 128
KP = 15
KPP = 16
NH = N * H
TW = 128
FH = 120
BN_EPS = 1e-5
NEG_SLOPE = 0.2

B = 400
WIN = 128


def _sc_gather(tab, idx_flat):
    mesh = plsc.VectorSubcoreMesh(core_axis_name="c", subcore_axis_name="s")

    @functools.partial(
        pl.kernel,
        out_type=jax.ShapeDtypeStruct((NH, TW), jnp.float32),
        mesh=mesh,
    )
    def k(tab_hbm, idx_hbm, g_hbm):
        def body(i_vmem, g_vmem):
            pltpu.sync_copy(tab_hbm.at[i_vmem.at[0]], g_vmem)

        pltpu.emit_pipeline(
            body,
            grid=(NH // WIN,),
            in_specs=[pl.BlockSpec((1, WIN), lambda i: (0, i))],
            out_specs=[pl.BlockSpec((WIN, TW), lambda i: (i, 0))],
            core_axis_name=("c", "s"),
            dimension_semantics=(pltpu.PARALLEL,),
        )(idx_hbm, g_hbm)

    return k(tab, idx_flat)


def _main_body(g_ref, q_ref, k_ref, w_ref, out_ref, stats_ref, acc_ref):
    i = pl.program_id(0)
    nb = pl.num_programs(0)
    gfull = g_ref[...]
    packed = lax.bitcast_convert_type(gfull[:, FH:FH + 4], jnp.uint32)
    f_hi = lax.bitcast_convert_type(
        packed & jnp.uint32(0xFFFF0000), jnp.float32)
    f_lo = lax.bitcast_convert_type(
        packed << 16, jnp.float32)
    g = jnp.concatenate([gfull[:, 0:FH], f_hi, f_lo], axis=1)
    gf = g.reshape(B, H, DIN)
    s4 = gfull[:, FH + 4:TW]
    q4 = q_ref[...][:, 0:4]
    qr = jnp.broadcast_to(q4[:, None, :], (B, H, 4)).reshape(B * H, 4)
    asub = s4[:, 0:3] - qr[:, 0:3]
    anorm = jnp.sum(asub * asub, axis=1, keepdims=True)
    k3 = k_ref[...]
    knorm = jnp.sum(k3 * k3, axis=1)
    dots = lax.dot_general(
        asub, k3, (((1,), (1,)), ((), ())),
        precision=lax.Precision.HIGHEST,
        preferred_element_type=jnp.float32)
    sq = anorm - 2.0 * dots + knorm[None, :]
    w = jnp.maximum(
        1.0 - jnp.sqrt(jnp.maximum(sq, 0.0)), 0.0).reshape(B, H, KPP)
    wf = lax.dot_general(
        w, gf, (((1,), (1,)), ((0,), (0,))),
        preferred_element_type=jnp.float32,
    )
    acc = jnp.zeros((B, DOUT), jnp.float32)
    for kk in range(KP):
        acc = acc + jnp.dot(wf[:, kk, :], w_ref[kk],
                            preferred_element_type=jnp.float32)
    out_ref[...] = acc

    @pl.when(i == 0)
    def _():
        acc_ref[...] = jnp.zeros_like(acc_ref)

    acc_ref[0:1, :] += jnp.sum(acc, axis=0, keepdims=True)
    acc_ref[1:2, :] += jnp.sum(acc * acc, axis=0, keepdims=True)

    @pl.when(i == nb - 1)
    def _():
        stats_ref[...] = acc_ref[...]


def _tc_main(gath, posq, k3p, W):
    nb = N // B
    return pl.pallas_call(
        _main_body,
        grid=(nb,),
        in_specs=[
            pl.BlockSpec((B * H, TW), lambda i: (i, 0)),
            pl.BlockSpec((B, 8), lambda i: (i, 0)),
            pl.BlockSpec((KPP, 3), lambda i: (0, 0)),
            pl.BlockSpec((KP, DIN, DOUT), lambda i: (0, 0, 0)),
        ],
        out_specs=[
            pl.BlockSpec((B, DOUT), lambda i: (i, 0)),
            pl.BlockSpec((8, DOUT), lambda i: (0, 0)),
        ],
        out_shape=[
            jax.ShapeDtypeStruct((N, DOUT), jnp.float32),
            jax.ShapeDtypeStruct((8, DOUT), jnp.float32),
        ],
        scratch_shapes=[pltpu.VMEM((8, DOUT), jnp.float32)],
    )(gath, posq, k3p, W)


def _bn_body(o_ref, st_ref, ga_ref, be_ref, out_ref):
    inv_n = 1.0 / N
    mean = st_ref[0:1, :] * inv_n
    var = st_ref[1:2, :] * inv_n - mean * mean
    inv = lax.rsqrt(var + BN_EPS)
    y = (o_ref[...] - mean) * (inv * ga_ref[...]) + be_ref[...]
    out_ref[...] = jnp.where(y >= 0.0, y, NEG_SLOPE * y)


def _tc_bn(out_raw, stats, gamma, beta):
    b2 = 2000
    return pl.pallas_call(
        _bn_body,
        grid=(N // b2,),
        in_specs=[
            pl.BlockSpec((b2, DOUT), lambda i: (i, 0)),
            pl.BlockSpec((8, DOUT), lambda i: (0, 0)),
            pl.BlockSpec((1, DOUT), lambda i: (0, 0)),
            pl.BlockSpec((1, DOUT), lambda i: (0, 0)),
        ],
        out_specs=pl.BlockSpec((b2, DOUT), lambda i: (i, 0)),
        out_shape=jax.ShapeDtypeStruct((N, DOUT), jnp.float32),
    )(out_raw, stats, gamma, beta)


def kernel(pos, x, idx_neighboors, K_points, W, bn_gamma, bn_beta):
    idx_flat = idx_neighboors.astype(jnp.int32).reshape(1, NH)
    pos_t = jnp.concatenate(
        [pos, jnp.full((1, 3), 1e6, pos.dtype)], axis=0)
    pnorm = jnp.sum(pos_t * pos_t, axis=1, keepdims=True)
    p4 = jnp.concatenate([pos_t, pnorm], axis=1)
    feats_t = jnp.concatenate(
        [x, jnp.zeros((1, DIN), x.dtype)], axis=0)
    t1 = lax.bitcast_convert_type(
        feats_t[:, FH:FH + 4].astype(jnp.bfloat16), jnp.uint16)
    t2 = lax.bitcast_convert_type(
        feats_t[:, FH + 4:DIN].astype(jnp.bfloat16), jnp.uint16)
    packed = (t1.astype(jnp.uint32) << 16) | t2.astype(jnp.uint32)
    tab = jnp.concatenate(
        [feats_t[:, 0:FH], lax.bitcast_convert_type(packed, jnp.float32), p4],
        axis=1)
    posq = jnp.pad(pos, ((0, 0), (0, 5)))
    k3p = jnp.concatenate(
        [K_points, jnp.full((KPP - KP, 3), 1e6, jnp.float32)], axis=0)

    gath = _sc_gather(tab, idx_flat)
    out_raw, stats = _tc_main(gath, posq, k3p, W)
    return _tc_bn(out_raw, stats, bn_gamma.reshape(1, DOUT),
                  bn_beta.reshape(1, DOUT))


# speedup vs baseline: 2.9246x; 1.0072x over previous
import functools

import jax
import jax.numpy as jnp
from jax import lax
from jax.experimental import pallas as pl
from jax.experimental.pallas import tpu as pltpu
from jax.experimental.pallas import tpu_sc as plsc

N = 10000
H = 32
DIN = 128
DOUT = 128
KP = 15
KPP = 16
NH = N * H
TW = 128
FH = 120
BN_EPS = 1e-5
NEG_SLOPE = 0.2

B = 400
WIN = 128


def _sc_gather(tab, idx_flat):
    mesh = plsc.VectorSubcoreMesh(core_axis_name="c", subcore_axis_name="s")

    @functools.partial(
        pl.kernel,
        out_type=jax.ShapeDtypeStruct((NH, TW), jnp.float32),
        mesh=mesh,
    )
    def k(tab_hbm, idx_hbm, g_hbm):
        def body(i_vmem, g_vmem):
            pltpu.sync_copy(tab_hbm.at[i_vmem.at[0]], g_vmem)

        pltpu.emit_pipeline(
            body,
            grid=(NH // WIN,),
            in_specs=[pl.BlockSpec((1, WIN), lambda i: (0, i))],
            out_specs=[pl.BlockSpec((WIN, TW), lambda i: (i, 0))],
            core_axis_name=("c", "s"),
            dimension_semantics=(pltpu.PARALLEL,),
        )(idx_hbm, g_hbm)

    return k(tab, idx_flat)


def _main_body(g_ref, q_ref, k_ref, w_ref, out_ref, stats_ref):
    gfull = g_ref[...]
    packed = lax.bitcast_convert_type(gfull[:, FH:FH + 4], jnp.uint32)
    f_hi = lax.bitcast_convert_type(
        packed & jnp.uint32(0xFFFF0000), jnp.float32)
    f_lo = lax.bitcast_convert_type(
        packed << 16, jnp.float32)
    g = jnp.concatenate([gfull[:, 0:FH], f_hi, f_lo], axis=1)
    gf = g.reshape(B, H, DIN)
    s4 = gfull[:, FH + 4:TW]
    q4 = q_ref[...][:, 0:4]
    qr = jnp.broadcast_to(q4[:, None, :], (B, H, 4)).reshape(B * H, 4)
    asub = s4[:, 0:3] - qr[:, 0:3]
    anorm = jnp.sum(asub * asub, axis=1, keepdims=True)
    k3 = k_ref[...]
    knorm = jnp.sum(k3 * k3, axis=1)
    dots = lax.dot_general(
        asub, k3, (((1,), (1,)), ((), ())),
        precision=lax.Precision.HIGHEST,
        preferred_element_type=jnp.float32)
    sq = anorm - 2.0 * dots + knorm[None, :]
    w = jnp.maximum(
        1.0 - jnp.sqrt(jnp.maximum(sq, 0.0)), 0.0).reshape(B, H, KPP)
    wf = lax.dot_general(
        w, gf, (((1,), (1,)), ((0,), (0,))),
        preferred_element_type=jnp.float32,
    )
    acc = jnp.zeros((B, DOUT), jnp.float32)
    for kk in range(KP):
        acc = acc + jnp.dot(wf[:, kk, :], w_ref[kk],
                            preferred_element_type=jnp.float32)
    out_ref[...] = acc

    stats_ref[0, 0:1, :] = jnp.sum(acc, axis=0, keepdims=True)
    stats_ref[0, 1:2, :] = jnp.sum(acc * acc, axis=0, keepdims=True)


def _tc_main(gath, posq, k3p, W):
    nb = N // B
    return pl.pallas_call(
        _main_body,
        grid=(nb,),
        in_specs=[
            pl.BlockSpec((B * H, TW), lambda i: (i, 0)),
            pl.BlockSpec((B, 8), lambda i: (i, 0)),
            pl.BlockSpec((KPP, 3), lambda i: (0, 0)),
            pl.BlockSpec((KP, DIN, DOUT), lambda i: (0, 0, 0)),
        ],
        out_specs=[
            pl.BlockSpec((B, DOUT), lambda i: (i, 0)),
            pl.BlockSpec((1, 2, DOUT), lambda i: (i, 0, 0)),
        ],
        out_shape=[
            jax.ShapeDtypeStruct((N, DOUT), jnp.float32),
            jax.ShapeDtypeStruct((nb, 2, DOUT), jnp.float32),
        ],
        compiler_params=pltpu.CompilerParams(
            dimension_semantics=("parallel",)),
    )(gath, posq, k3p, W)


def _bn_body(o_ref, st_ref, ga_ref, be_ref, out_ref):
    inv_n = 1.0 / N
    mean = jnp.sum(st_ref[:, 0, :], axis=0, keepdims=True) * inv_n
    var = (jnp.sum(st_ref[:, 1, :], axis=0, keepdims=True) * inv_n
           - mean * mean)
    inv = lax.rsqrt(var + BN_EPS)
    y = (o_ref[...] - mean) * (inv * ga_ref[...]) + be_ref[...]
    out_ref[...] = jnp.where(y >= 0.0, y, NEG_SLOPE * y)


def _tc_bn(out_raw, stats, gamma, beta):
    b2 = 2000
    return pl.pallas_call(
        _bn_body,
        grid=(N // b2,),
        in_specs=[
            pl.BlockSpec((b2, DOUT), lambda i: (i, 0)),
            pl.BlockSpec((N // B, 2, DOUT), lambda i: (0, 0, 0)),
            pl.BlockSpec((1, DOUT), lambda i: (0, 0)),
            pl.BlockSpec((1, DOUT), lambda i: (0, 0)),
        ],
        out_specs=pl.BlockSpec((b2, DOUT), lambda i: (i, 0)),
        out_shape=jax.ShapeDtypeStruct((N, DOUT), jnp.float32),
    )(out_raw, stats, gamma, beta)


def kernel(pos, x, idx_neighboors, K_points, W, bn_gamma, bn_beta):
    idx_flat = idx_neighboors.astype(jnp.int32).reshape(1, NH)
    pos_t = jnp.concatenate(
        [pos, jnp.full((1, 3), 1e6, pos.dtype)], axis=0)
    pnorm = jnp.sum(pos_t * pos_t, axis=1, keepdims=True)
    p4 = jnp.concatenate([pos_t, pnorm], axis=1)
    feats_t = jnp.concatenate(
        [x, jnp.zeros((1, DIN), x.dtype)], axis=0)
    t1 = lax.bitcast_convert_type(
        feats_t[:, FH:FH + 4].astype(jnp.bfloat16), jnp.uint16)
    t2 = lax.bitcast_convert_type(
        feats_t[:, FH + 4:DIN].astype(jnp.bfloat16), jnp.uint16)
    packed = (t1.astype(jnp.uint32) << 16) | t2.astype(jnp.uint32)
    tab = jnp.concatenate(
        [feats_t[:, 0:FH], lax.bitcast_convert_type(packed, jnp.float32), p4],
        axis=1)
    posq = jnp.pad(pos, ((0, 0), (0, 5)))
    k3p = jnp.concatenate(
        [K_points, jnp.full((KPP - KP, 3), 1e6, jnp.float32)], axis=0)

    gath = _sc_gather(tab, idx_flat)
    out_raw, stats = _tc_main(gath, posq, k3p, W)
    return _tc_bn(out_raw, stats, bn_gamma.reshape(1, DOUT),
                  bn_beta.reshape(1, DOUT))
